# baseline (speedup 1.0000x reference)
import dataclasses
import functools
import jax
import jax.numpy as jnp
from jax import lax
from jax.experimental import pallas as pl
from jax.experimental.pallas import tpu as pltpu
from jax.experimental.pallas import tpu_sc as plsc

B, C, H, W, M, K = 2, 64, 224, 224, 8192, 9
C2 = 2 * C
NC, NS = 2, 16
NW = NC * NS
HW = H * W
WP = W + 8
TH = 8
T = H // TH
ROWS = (T + 1) * TH


def _taps(xa_ref, xb_ref, w_ref, y):
    xcat = jnp.concatenate([xa_ref[0], xb_ref[0, :3]], axis=0).reshape(-1, C)
    nrow = TH * WP
    for dy in range(3):
        for dx in range(3):
            off = dy * WP + dx
            y = y + jnp.dot(xcat[off:off + nrow, :], w_ref[dy * 3 + dx],
                            preferred_element_type=jnp.float32)
    return y


def _conv_quad_body(da_ref, db_ref, ra_ref, rb_ref, wd_ref, wr_ref, b_ref,
                    m1_ref, dr0_ref, dr1_ref, base_ref):
    nrow = TH * WP
    y = jnp.zeros((nrow, 2 * C2), jnp.float32)
    y = _taps(da_ref, db_ref, wd_ref, y)
    y = _taps(ra_ref, rb_ref, wr_ref, y)
    y3 = y.reshape(TH, WP, 2 * C2)[:, :W, :] + b_ref[0]
    dr0 = jax.nn.relu(y3[:, :, :C2])
    dr0_ref[0] = dr0
    dr1_ref[0] = y3[:, :, C2:]
    base_ref[0] = m1_ref[0] * dr0


def _conv_quad(xpd, xpr, wd, wr, b4, m1):
    out = jax.ShapeDtypeStruct((B, H, W, C2), jnp.float32)
    blkA = lambda b, t: (b, t, 0, 0)
    blkB = lambda b, t: (b, t + 1, 0, 0)
    fix3 = lambda b, t: (0, 0, 0)
    return pl.pallas_call(
        _conv_quad_body,
        grid=(B, T),
        in_specs=[
            pl.BlockSpec((1, TH, WP, C), blkA),
            pl.BlockSpec((1, TH, WP, C), blkB),
            pl.BlockSpec((1, TH, WP, C), blkA),
            pl.BlockSpec((1, TH, WP, C), blkB),
            pl.BlockSpec((9, C, 2 * C2), fix3),
            pl.BlockSpec((9, C, 2 * C2), fix3),
            pl.BlockSpec((1, 2 * C2), lambda b, t: (0, 0)),
            pl.BlockSpec((1, TH, W, C2), blkA),
        ],
        out_specs=[
            pl.BlockSpec((1, TH, W, C2), blkA),
            pl.BlockSpec((1, TH, W, C2), blkA),
            pl.BlockSpec((1, TH, W, C2), blkA),
        ],
        out_shape=[out, out, out],
    )(xpd, xpd, xpr, xpr, wd, wr, b4, m1)


def _conv_final_body(xa_ref, xb_ref, w_ref, b_ref, x1_ref, o_ref):
    nrow = TH * WP
    xcat = jnp.concatenate([xa_ref[0], xb_ref[0, :3]], axis=0).reshape(-1, C2)
    y = jnp.zeros((nrow, C2), jnp.float32)
    for dy in range(3):
        for dx in range(3):
            off = dy * WP + dx
            y = y + jnp.dot(xcat[off:off + nrow, :], w_ref[dy * 3 + dx],
                            preferred_element_type=jnp.float32)
    y3 = y.reshape(TH, WP, C2)[:, :W, :]
    o_ref[0] = jax.nn.relu(y3 + b_ref[0] + x1_ref[0])


def _conv_final(xp, w9, b1, x1):
    return pl.pallas_call(
        _conv_final_body,
        grid=(B, T),
        in_specs=[
            pl.BlockSpec((1, TH, WP, C2), lambda b, t: (b, t, 0, 0)),
            pl.BlockSpec((1, TH, WP, C2), lambda b, t: (b, t + 1, 0, 0)),
            pl.BlockSpec((9, C2, C2), lambda b, t: (0, 0, 0)),
            pl.BlockSpec((1, C2), lambda b, t: (0, 0)),
            pl.BlockSpec((1, TH, W, C2), lambda b, t: (b, t, 0, 0)),
        ],
        out_specs=pl.BlockSpec((1, TH, W, C2), lambda b, t: (b, t, 0, 0)),
        out_shape=jax.ShapeDtypeStruct((B, H, W, C2), jnp.float32),
    )(xp, xp, w9, b1, x1)


def _sc_gather(table, idx, chunk):
    nidx = idx.shape[0]
    per_w = nidx // NW
    nch = per_w // chunk
    mesh = plsc.VectorSubcoreMesh(core_axis_name="c", subcore_axis_name="s")

    @functools.partial(
        pl.kernel, mesh=mesh,
        out_type=jax.ShapeDtypeStruct((nidx, C2), jnp.float32),
        scratch_types=[
            pltpu.VMEM((chunk,), jnp.int32),
            pltpu.VMEM((chunk, C2), jnp.float32),
            pltpu.SemaphoreType.DMA,
        ],
    )
    def k(table_hbm, idx_hbm, out_hbm, idx_v, rows_v, sem):
        wid = lax.axis_index("s") * NC + lax.axis_index("c")
        base = wid * per_w

        @pl.loop(0, nch)
        def _(j):
            off = base + j * chunk
            pltpu.sync_copy(idx_hbm.at[pl.ds(off, chunk)], idx_v)
            pltpu.async_copy(table_hbm.at[idx_v], rows_v, sem).wait()
            pltpu.sync_copy(rows_v, out_hbm.at[pl.ds(off, chunk)])

    return k(table, idx)


OWN = B * HW // NW
SCH = 256
NCH = (OWN + SCH - 1) // SCH + 1
NG = B * M // 16


def _sc_scatter(gid, fdr):
    mesh = plsc.VectorSubcoreMesh(core_axis_name="c", subcore_axis_name="s")
    dump = B * HW
    cp = pltpu.CompilerParams()
    if "needs_layout_passes" in pltpu.CompilerParams.__dataclass_fields__:
        cp = dataclasses.replace(cp, needs_layout_passes=False)

    @functools.partial(
        pl.kernel, mesh=mesh, compiler_params=cp,
        out_type=jax.ShapeDtypeStruct((B * HW + 8, C2), jnp.float32),
        scratch_types=[
            pltpu.VMEM((B * M,), jnp.int32),
            pltpu.VMEM((OWN,), jnp.int32),
            pltpu.VMEM((OWN + SCH + 16,), jnp.int32),
            pltpu.VMEM((OWN + SCH + 16,), jnp.int32),
            pltpu.VMEM((SCH, C2), jnp.float32),
            pltpu.VMEM((SCH, C2), jnp.float32),
            pltpu.SemaphoreType.DMA,
        ],
    )
    def k(gid_hbm, fdr_hbm, delta_hbm, gidv, winv, cg, cm, rows, zrows,
          sem):
        wid = lax.axis_index("s") * NC + lax.axis_index("c")
        lo = wid * OWN
        hi = lo + OWN
        z16 = jnp.zeros((16,), jnp.float32)
        i16 = lax.iota(jnp.int32, 16)

        pltpu.sync_copy(gid_hbm, gidv)

        @pl.loop(0, SCH * C2 // 16)
        def _(i):
            zrows.at[i // (C2 // 16)][pl.ds((i % (C2 // 16)) * 16, 16)] = z16

        @pl.loop(0, OWN // SCH)
        def _(j):
            pltpu.sync_copy(zrows, delta_hbm.at[pl.ds(lo + j * SCH, SCH)])

        if OWN % SCH:
            pltpu.sync_copy(zrows.at[pl.ds(0, OWN % SCH)],
                            delta_hbm.at[pl.ds(lo + OWN - OWN % SCH,
                                               OWN % SCH)])

        @pl.when(wid == 0)
        def _():
            pltpu.sync_copy(zrows.at[pl.ds(0, 8)],
                            delta_hbm.at[pl.ds(dump, 8)])

        @pl.loop(0, OWN // 16)
        def _(i):
            winv[pl.ds(i * 16, 16)] = jnp.full((16,), -1, jnp.int32)

        @pl.loop(0, NG)
        def _(g):
            gv = gidv[pl.ds(g * 16, 16)]
            inr = (gv >= lo) & (gv < hi)
            loc = jnp.where(inr, gv - lo, 0)
            mids = g * 16 + i16
            plsc.store_scatter(winv, [loc], mids, mask=inr)

        def pass2(g, c):
            gv = gidv[pl.ds(g * 16, 16)]
            inr = (gv >= lo) & (gv < hi)
            loc = jnp.where(inr, gv - lo, 0)
            mids = g * 16 + i16
            win16 = plsc.load_gather(winv, [loc])
            kept = inr & (win16 == mids)
            plsc.store_compressed(cg.at[pl.ds(c, 16)], gv, mask=kept)
            plsc.store_compressed(cm.at[pl.ds(c, 16)], mids, mask=kept)
            return c + jnp.sum(kept.astype(jnp.int32))

        c = lax.fori_loop(0, NG, pass2, 0)

        ones = i16 >= 0

        @pl.loop(0, SCH // 16)
        def _(i):
            plsc.store_compressed(cg.at[pl.ds(c + i * 16, 16)],
                                  jnp.full((16,), dump, jnp.int32), mask=ones)
            plsc.store_compressed(cm.at[pl.ds(c + i * 16, 16)],
                                  jnp.zeros((16,), jnp.int32), mask=ones)

        nch = (c + SCH - 1) // SCH

        @pl.loop(0, NCH)
        def _(j):
            @pl.when(j < nch)
            def _():
                pltpu.async_copy(fdr_hbm.at[cm.at[pl.ds(j * SCH, SCH)]],
                                 rows, sem).wait()

                @pl.loop(0, SCH // 16)
                def _(kk):
                    gvec = cg[pl.ds(j * SCH + kk * 16, 16)]
                    pltpu.async_copy(rows.at[pl.ds(kk * 16, 16)],
                                     delta_hbm.at[gvec], sem).wait()

    return k(gid, fdr)


def _mlp_body(kf_ref, dr_ref, waff_ref, wself_ref, baff_ref,
              w1_ref, b1_ref, w34_ref, b34_ref, w56_ref, b56_ref, o_ref):
    dot = functools.partial(jnp.dot, preferred_element_type=jnp.float32)
    dr_new = jax.nn.relu(dot(kf_ref[...], waff_ref[...])
                         + dot(dr_ref[...], wself_ref[...]) + baff_ref[...])
    fuse = jax.nn.relu(dot(dr_new, w1_ref[...]) + b1_ref[...])
    att = jax.nn.sigmoid(dot(fuse, w34_ref[...]) + b34_ref[...])
    att_pack = jnp.concatenate(
        [jnp.broadcast_to(att[:, 0:1], att.shape[:1] + (C,)),
         jnp.broadcast_to(att[:, 1:2], att.shape[:1] + (C,))], axis=1)
    dr_sw = jnp.concatenate([dr_new[:, C:], dr_new[:, :C]], axis=1)
    impt = dr_new + dr_sw * att_pack
    o_ref[...] = jax.nn.relu(dot(impt, w56_ref[...]) + b56_ref[...])


def _mlp(kf, dr_dis, waff, wself, baff, w1, b1, w34, b34, w56, b56, TM=2048):
    n = dr_dis.shape[0]
    row = lambda i: (i, 0)
    fix = lambda i: (0, 0)
    return pl.pallas_call(
        _mlp_body,
        grid=(n // TM,),
        in_specs=[
            pl.BlockSpec((TM, K * C2), row), pl.BlockSpec((TM, C2), row),
            pl.BlockSpec((K * C2, C2), fix), pl.BlockSpec((C2, C2), fix),
            pl.BlockSpec((1, C2), fix),
            pl.BlockSpec((C2, C), fix), pl.BlockSpec((1, C), fix),
            pl.BlockSpec((C, 2), fix), pl.BlockSpec((1, 2), fix),
            pl.BlockSpec((C2, C2), fix), pl.BlockSpec((1, C2), fix),
        ],
        out_specs=pl.BlockSpec((TM, C2), row),
        out_shape=jax.ShapeDtypeStruct((n, C2), jnp.float32),
    )(kf, dr_dis, waff, wself, baff, w1, b1, w34, b34, w56, b56)


def _pad_flat(x_nhwc):
    return jnp.pad(x_nhwc, ((0, 0), (1, ROWS - H - 1), (1, WP - W - 1), (0, 0)))


def _w9(w_oihw):
    return w_oihw.transpose(2, 3, 1, 0).reshape(9, C, -1)


def _blkdiag(a, b):
    ka, na = a.shape
    kb, nb = b.shape
    z = jnp.zeros((ka + kb, na + nb), a.dtype)
    return z.at[:ka, :na].set(a).at[ka:, na:].set(b)


def kernel(d_feat, r_feat, masks, w_d0, b_d0, w_d1, b_d1, w_d2, b_d2, w_r0, b_r0, w_r1, b_r1, w_r2, b_r2, w_affd, b_affd, w_affr, b_affr, w_fc1, b_fc1, w_fc3, b_fc3, w_fc4, b_fc4, w_fc5, b_fc5, w_fc6, b_fc6, locs, nnidxs):
    d_nhwc = d_feat.transpose(0, 2, 3, 1)
    r_nhwc = r_feat.transpose(0, 2, 3, 1)
    m1 = jnp.broadcast_to((1.0 - masks).transpose(0, 2, 3, 1), (B, H, W, C2))

    zc = jnp.zeros((9, C, C), jnp.float32)
    wd = jnp.concatenate([_w9(w_d0), zc, _w9(w_d1), zc], axis=-1)
    wr = jnp.concatenate([zc, _w9(w_r0), zc, _w9(w_r1)], axis=-1)
    b4 = jnp.concatenate([b_d0, b_r0, b_d1, b_r1])[None, :]

    dr0, dr1, base = _conv_quad(_pad_flat(d_nhwc), _pad_flat(r_nhwc),
                                wd, wr, b4, m1)

    g = locs[:, :, 0].astype(jnp.int32) * W + locs[:, :, 1].astype(jnp.int32)
    gid = (g + jnp.arange(B, dtype=jnp.int32)[:, None] * HW).reshape(-1)
    nng = (nnidxs.astype(jnp.int32)
           + jnp.arange(B, dtype=jnp.int32)[:, None, None] * M).reshape(-1)

    dr_dis = _sc_gather(dr0.reshape(B * HW, C2), gid, 512)
    kf = _sc_gather(dr_dis, nng, 512).reshape(B * M, K * C2)

    wa3 = _w9_aff(w_affd, w_affr)
    wself = _blkdiag(w_affd.T[K * C:], w_affr.T[K * C:])
    baff = jnp.concatenate([b_affd, b_affr])[None, :]
    w34 = jnp.concatenate([w_fc3, w_fc4], axis=0).T
    b34 = jnp.concatenate([b_fc3, b_fc4])[None, :]
    w56 = _blkdiag(w_fc5.T, w_fc6.T)
    b56 = jnp.concatenate([b_fc5, b_fc6])[None, :]

    fdr = _mlp(kf, dr_dis, wa3, wself, baff, w_fc1.T, b_fc1[None, :],
               w34, b34, w56, b56)

    delta = _sc_scatter(gid, fdr)[:B * HW]

    xb = (base.reshape(B * HW, C2) + delta).reshape(B, H, W, C2)

    w2 = jnp.zeros((9, C2, C2), jnp.float32)
    w2 = w2.at[:, :C, :C].set(_w9(w_d2)).at[:, C:, C:].set(_w9(w_r2))
    b2 = jnp.concatenate([b_d2, b_r2])[None, :]
    out = _conv_final(_pad_flat(xb), w2, b2, dr1)
    return (out[..., :C].transpose(0, 3, 1, 2),
            out[..., C:].transpose(0, 3, 1, 2))


def _w9_aff(w_affd, w_affr):
    wad = w_affd.T[:K * C].reshape(K, C, C)
    war = w_affr.T[:K * C].reshape(K, C, C)
    z = jnp.zeros((K, C, C), jnp.float32)
    top = jnp.concatenate([wad, z], axis=-1)
    bot = jnp.concatenate([z, war], axis=-1)
    return jnp.concatenate([top, bot], axis=1).reshape(K * C2, C2)


# speedup vs baseline: 1.0164x; 1.0164x over previous
import dataclasses
import functools
import jax
import jax.numpy as jnp
from jax import lax
from jax.experimental import pallas as pl
from jax.experimental.pallas import tpu as pltpu
from jax.experimental.pallas import tpu_sc as plsc

B, C, H, W, M, K = 2, 64, 224, 224, 8192, 9
C2 = 2 * C
NC, NS = 2, 16
NW = NC * NS
HW = H * W
WP = W + 8
TH = 8
T = H // TH
ROWS = (T + 1) * TH


def _taps(xa_ref, xb_ref, w_ref, y):
    xcat = jnp.concatenate([xa_ref[0], xb_ref[0, :3]], axis=0).reshape(-1, C)
    nrow = TH * WP
    for dy in range(3):
        for dx in range(3):
            off = dy * WP + dx
            y = y + jnp.dot(xcat[off:off + nrow, :], w_ref[dy * 3 + dx],
                            preferred_element_type=jnp.float32)
    return y


def _conv_quad_body(da_ref, db_ref, ra_ref, rb_ref, wd_ref, wr_ref, b_ref,
                    m1_ref, dr0_ref, dr1_ref, base_ref):
    nrow = TH * WP
    y = jnp.zeros((nrow, 2 * C2), jnp.float32)
    y = _taps(da_ref, db_ref, wd_ref, y)
    y = _taps(ra_ref, rb_ref, wr_ref, y)
    y3 = y.reshape(TH, WP, 2 * C2)[:, :W, :] + b_ref[0]
    dr0 = jax.nn.relu(y3[:, :, :C2])
    dr0_ref[0] = dr0
    dr1_ref[0] = y3[:, :, C2:]
    base_ref[0] = m1_ref[0] * dr0


def _conv_quad(xpd, xpr, wd, wr, b4, m1):
    out = jax.ShapeDtypeStruct((B, H, W, C2), jnp.float32)
    blkA = lambda b, t: (b, t, 0, 0)
    blkB = lambda b, t: (b, t + 1, 0, 0)
    fix3 = lambda b, t: (0, 0, 0)
    return pl.pallas_call(
        _conv_quad_body,
        grid=(B, T),
        in_specs=[
            pl.BlockSpec((1, TH, WP, C), blkA),
            pl.BlockSpec((1, TH, WP, C), blkB),
            pl.BlockSpec((1, TH, WP, C), blkA),
            pl.BlockSpec((1, TH, WP, C), blkB),
            pl.BlockSpec((9, C, 2 * C2), fix3),
            pl.BlockSpec((9, C, 2 * C2), fix3),
            pl.BlockSpec((1, 2 * C2), lambda b, t: (0, 0)),
            pl.BlockSpec((1, TH, W, C2), blkA),
        ],
        out_specs=[
            pl.BlockSpec((1, TH, W, C2), blkA),
            pl.BlockSpec((1, TH, W, C2), blkA),
            pl.BlockSpec((1, TH, W, C2), blkA),
        ],
        out_shape=[out, out, out],
    )(xpd, xpd, xpr, xpr, wd, wr, b4, m1)


def _conv_final_body(xa_ref, xb_ref, w_ref, b_ref, x1_ref, o_ref):
    nrow = TH * WP
    xcat = jnp.concatenate([xa_ref[0], xb_ref[0, :3]], axis=0).reshape(-1, C2)
    y = jnp.zeros((nrow, C2), jnp.float32)
    for dy in range(3):
        for dx in range(3):
            off = dy * WP + dx
            y = y + jnp.dot(xcat[off:off + nrow, :], w_ref[dy * 3 + dx],
                            preferred_element_type=jnp.float32)
    y3 = y.reshape(TH, WP, C2)[:, :W, :]
    o_ref[0] = jax.nn.relu(y3 + b_ref[0] + x1_ref[0])


def _conv_final(xp, w9, b1, x1):
    return pl.pallas_call(
        _conv_final_body,
        grid=(B, T),
        in_specs=[
            pl.BlockSpec((1, TH, WP, C2), lambda b, t: (b, t, 0, 0)),
            pl.BlockSpec((1, TH, WP, C2), lambda b, t: (b, t + 1, 0, 0)),
            pl.BlockSpec((9, C2, C2), lambda b, t: (0, 0, 0)),
            pl.BlockSpec((1, C2), lambda b, t: (0, 0)),
            pl.BlockSpec((1, TH, W, C2), lambda b, t: (b, t, 0, 0)),
        ],
        out_specs=pl.BlockSpec((1, TH, W, C2), lambda b, t: (b, t, 0, 0)),
        out_shape=jax.ShapeDtypeStruct((B, H, W, C2), jnp.float32),
    )(xp, xp, w9, b1, x1)


def _sc_gather(table, idx, chunk):
    nidx = idx.shape[0]
    per_w = nidx // NW
    nch = per_w // chunk
    mesh = plsc.VectorSubcoreMesh(core_axis_name="c", subcore_axis_name="s")

    @functools.partial(
        pl.kernel, mesh=mesh,
        out_type=jax.ShapeDtypeStruct((nidx, C2), jnp.float32),
        scratch_types=[
            pltpu.VMEM((chunk,), jnp.int32),
            pltpu.VMEM((chunk, C2), jnp.float32),
            pltpu.SemaphoreType.DMA,
        ],
    )
    def k(table_hbm, idx_hbm, out_hbm, idx_v, rows_v, sem):
        wid = lax.axis_index("s") * NC + lax.axis_index("c")
        base = wid * per_w

        @pl.loop(0, nch)
        def _(j):
            off = base + j * chunk
            pltpu.sync_copy(idx_hbm.at[pl.ds(off, chunk)], idx_v)
            pltpu.async_copy(table_hbm.at[idx_v], rows_v, sem).wait()
            pltpu.sync_copy(rows_v, out_hbm.at[pl.ds(off, chunk)])

    return k(table, idx)


OWN = B * HW // NW
SCH = 256
NCH = (OWN + SCH - 1) // SCH + 1
NG = B * M // 16


def _sc_scatter(gid, fdr):
    mesh = plsc.VectorSubcoreMesh(core_axis_name="c", subcore_axis_name="s")
    dump = B * HW
    cp = pltpu.CompilerParams()
    if "needs_layout_passes" in pltpu.CompilerParams.__dataclass_fields__:
        cp = dataclasses.replace(cp, needs_layout_passes=False)

    @functools.partial(
        pl.kernel, mesh=mesh, compiler_params=cp,
        out_type=jax.ShapeDtypeStruct((B * HW + 8, C2), jnp.float32),
        scratch_types=[
            pltpu.VMEM((B * M,), jnp.int32),
            pltpu.VMEM((OWN,), jnp.int32),
            pltpu.VMEM((OWN + SCH + 16,), jnp.int32),
            pltpu.VMEM((OWN + SCH + 16,), jnp.int32),
            pltpu.VMEM((SCH, C2), jnp.float32),
            pltpu.VMEM((SCH, C2), jnp.float32),
            pltpu.SemaphoreType.DMA,
        ],
    )
    def k(gid_hbm, fdr_hbm, delta_hbm, gidv, winv, cg, cm, rows, zrows,
          sem):
        wid = lax.axis_index("s") * NC + lax.axis_index("c")
        lo = wid * OWN
        hi = lo + OWN
        z16 = jnp.zeros((16,), jnp.float32)
        i16 = lax.iota(jnp.int32, 16)

        pltpu.sync_copy(gid_hbm, gidv)

        @pl.loop(0, SCH * C2 // 16)
        def _(i):
            zrows.at[i // (C2 // 16)][pl.ds((i % (C2 // 16)) * 16, 16)] = z16

        zcopies = [
            pltpu.async_copy(zrows, delta_hbm.at[pl.ds(lo + j * SCH, SCH)],
                             sem)
            for j in range(OWN // SCH)
        ]
        if OWN % SCH:
            zcopies.append(pltpu.async_copy(
                zrows.at[pl.ds(0, OWN % SCH)],
                delta_hbm.at[pl.ds(lo + OWN - OWN % SCH, OWN % SCH)], sem))

        @pl.when(wid == 0)
        def _():
            pltpu.sync_copy(zrows.at[pl.ds(0, 8)],
                            delta_hbm.at[pl.ds(dump, 8)])

        @pl.loop(0, OWN // 16)
        def _(i):
            winv[pl.ds(i * 16, 16)] = jnp.full((16,), -1, jnp.int32)

        @pl.loop(0, NG)
        def _(g):
            gv = gidv[pl.ds(g * 16, 16)]
            inr = (gv >= lo) & (gv < hi)
            loc = jnp.where(inr, gv - lo, 0)
            mids = g * 16 + i16
            plsc.store_scatter(winv, [loc], mids, mask=inr)

        def pass2(g, c):
            gv = gidv[pl.ds(g * 16, 16)]
            inr = (gv >= lo) & (gv < hi)
            loc = jnp.where(inr, gv - lo, 0)
            mids = g * 16 + i16
            win16 = plsc.load_gather(winv, [loc])
            kept = inr & (win16 == mids)
            plsc.store_compressed(cg.at[pl.ds(c, 16)], gv, mask=kept)
            plsc.store_compressed(cm.at[pl.ds(c, 16)], mids, mask=kept)
            return c + jnp.sum(kept.astype(jnp.int32))

        c = lax.fori_loop(0, NG, pass2, 0)

        ones = i16 >= 0

        @pl.loop(0, SCH // 16)
        def _(i):
            plsc.store_compressed(cg.at[pl.ds(c + i * 16, 16)],
                                  jnp.full((16,), dump, jnp.int32), mask=ones)
            plsc.store_compressed(cm.at[pl.ds(c + i * 16, 16)],
                                  jnp.zeros((16,), jnp.int32), mask=ones)

        nch = (c + SCH - 1) // SCH

        for zc in zcopies:
            zc.wait()

        @pl.loop(0, NCH)
        def _(j):
            @pl.when(j < nch)
            def _():
                pltpu.async_copy(fdr_hbm.at[cm.at[pl.ds(j * SCH, SCH)]],
                                 rows, sem).wait()

                scs = []
                for kk in range(SCH // 16):
                    gvec = cg[pl.ds(j * SCH + kk * 16, 16)]
                    scs.append(pltpu.async_copy(rows.at[pl.ds(kk * 16, 16)],
                                                delta_hbm.at[gvec], sem))
                for h in scs:
                    h.wait()

    return k(gid, fdr)


def _mlp_body(kf_ref, dr_ref, waff_ref, wself_ref, baff_ref,
              w1_ref, b1_ref, w34_ref, b34_ref, w56_ref, b56_ref, o_ref):
    dot = functools.partial(jnp.dot, preferred_element_type=jnp.float32)
    dr_new = jax.nn.relu(dot(kf_ref[...], waff_ref[...])
                         + dot(dr_ref[...], wself_ref[...]) + baff_ref[...])
    fuse = jax.nn.relu(dot(dr_new, w1_ref[...]) + b1_ref[...])
    att = jax.nn.sigmoid(dot(fuse, w34_ref[...]) + b34_ref[...])
    att_pack = jnp.concatenate(
        [jnp.broadcast_to(att[:, 0:1], att.shape[:1] + (C,)),
         jnp.broadcast_to(att[:, 1:2], att.shape[:1] + (C,))], axis=1)
    dr_sw = jnp.concatenate([dr_new[:, C:], dr_new[:, :C]], axis=1)
    impt = dr_new + dr_sw * att_pack
    o_ref[...] = jax.nn.relu(dot(impt, w56_ref[...]) + b56_ref[...])


def _mlp(kf, dr_dis, waff, wself, baff, w1, b1, w34, b34, w56, b56, TM=2048):
    n = dr_dis.shape[0]
    row = lambda i: (i, 0)
    fix = lambda i: (0, 0)
    return pl.pallas_call(
        _mlp_body,
        grid=(n // TM,),
        in_specs=[
            pl.BlockSpec((TM, K * C2), row), pl.BlockSpec((TM, C2), row),
            pl.BlockSpec((K * C2, C2), fix), pl.BlockSpec((C2, C2), fix),
            pl.BlockSpec((1, C2), fix),
            pl.BlockSpec((C2, C), fix), pl.BlockSpec((1, C), fix),
            pl.BlockSpec((C, 2), fix), pl.BlockSpec((1, 2), fix),
            pl.BlockSpec((C2, C2), fix), pl.BlockSpec((1, C2), fix),
        ],
        out_specs=pl.BlockSpec((TM, C2), row),
        out_shape=jax.ShapeDtypeStruct((n, C2), jnp.float32),
    )(kf, dr_dis, waff, wself, baff, w1, b1, w34, b34, w56, b56)


def _pad_flat(x_nhwc):
    return jnp.pad(x_nhwc, ((0, 0), (1, ROWS - H - 1), (1, WP - W - 1), (0, 0)))


def _w9(w_oihw):
    return w_oihw.transpose(2, 3, 1, 0).reshape(9, C, -1)


def _blkdiag(a, b):
    ka, na = a.shape
    kb, nb = b.shape
    z = jnp.zeros((ka + kb, na + nb), a.dtype)
    return z.at[:ka, :na].set(a).at[ka:, na:].set(b)


def kernel(d_feat, r_feat, masks, w_d0, b_d0, w_d1, b_d1, w_d2, b_d2, w_r0, b_r0, w_r1, b_r1, w_r2, b_r2, w_affd, b_affd, w_affr, b_affr, w_fc1, b_fc1, w_fc3, b_fc3, w_fc4, b_fc4, w_fc5, b_fc5, w_fc6, b_fc6, locs, nnidxs):
    d_nhwc = d_feat.transpose(0, 2, 3, 1)
    r_nhwc = r_feat.transpose(0, 2, 3, 1)
    m1 = jnp.broadcast_to((1.0 - masks).transpose(0, 2, 3, 1), (B, H, W, C2))

    zc = jnp.zeros((9, C, C), jnp.float32)
    wd = jnp.concatenate([_w9(w_d0), zc, _w9(w_d1), zc], axis=-1)
    wr = jnp.concatenate([zc, _w9(w_r0), zc, _w9(w_r1)], axis=-1)
    b4 = jnp.concatenate([b_d0, b_r0, b_d1, b_r1])[None, :]

    dr0, dr1, base = _conv_quad(_pad_flat(d_nhwc), _pad_flat(r_nhwc),
                                wd, wr, b4, m1)

    g = locs[:, :, 0].astype(jnp.int32) * W + locs[:, :, 1].astype(jnp.int32)
    gid = (g + jnp.arange(B, dtype=jnp.int32)[:, None] * HW).reshape(-1)
    nng = (nnidxs.astype(jnp.int32)
           + jnp.arange(B, dtype=jnp.int32)[:, None, None] * M).reshape(-1)

    dr_dis = _sc_gather(dr0.reshape(B * HW, C2), gid, 512)
    kf = _sc_gather(dr_dis, nng, 512).reshape(B * M, K * C2)

    wa3 = _w9_aff(w_affd, w_affr)
    wself = _blkdiag(w_affd.T[K * C:], w_affr.T[K * C:])
    baff = jnp.concatenate([b_affd, b_affr])[None, :]
    w34 = jnp.concatenate([w_fc3, w_fc4], axis=0).T
    b34 = jnp.concatenate([b_fc3, b_fc4])[None, :]
    w56 = _blkdiag(w_fc5.T, w_fc6.T)
    b56 = jnp.concatenate([b_fc5, b_fc6])[None, :]

    fdr = _mlp(kf, dr_dis, wa3, wself, baff, w_fc1.T, b_fc1[None, :],
               w34, b34, w56, b56)

    delta = _sc_scatter(gid, fdr)[:B * HW]

    xb = (base.reshape(B * HW, C2) + delta).reshape(B, H, W, C2)

    w2 = jnp.zeros((9, C2, C2), jnp.float32)
    w2 = w2.at[:, :C, :C].set(_w9(w_d2)).at[:, C:, C:].set(_w9(w_r2))
    b2 = jnp.concatenate([b_d2, b_r2])[None, :]
    out = _conv_final(_pad_flat(xb), w2, b2, dr1)
    return (out[..., :C].transpose(0, 3, 1, 2),
            out[..., C:].transpose(0, 3, 1, 2))


def _w9_aff(w_affd, w_affr):
    wad = w_affd.T[:K * C].reshape(K, C, C)
    war = w_affr.T[:K * C].reshape(K, C, C)
    z = jnp.zeros((K, C, C), jnp.float32)
    top = jnp.concatenate([wad, z], axis=-1)
    bot = jnp.concatenate([z, war], axis=-1)
    return jnp.concatenate([top, bot], axis=1).reshape(K * C2, C2)
